# baseline (speedup 1.0000x reference)
import jax
from jax.experimental import pallas as pl


_SEQ_BLOCK = 512


def _add_block(tensor_ref, pos_ref, out_ref):
    i = pl.program_id(0)
    out_ref[...] = tensor_ref[...] + pos_ref[pl.ds(i * _SEQ_BLOCK, _SEQ_BLOCK), :]


def kernel(tensor_in, pos_table):
    batch, seq, dim = tensor_in.shape
    grid = (seq // _SEQ_BLOCK, batch)
    return pl.pallas_call(
        _add_block,
        grid=grid,
        in_specs=[
            pl.BlockSpec((1, _SEQ_BLOCK, dim), lambda i, j: (j, i, 0)),
            pl.BlockSpec((seq, dim), lambda i, j: (0, 0)),
        ],
        out_specs=pl.BlockSpec((1, _SEQ_BLOCK, dim), lambda i, j: (j, i, 0)),
        out_shape=jax.ShapeDtypeStruct(tensor_in.shape, tensor_in.dtype),
    )(tensor_in, pos_table)


# speedup vs baseline: 1.0627x; 1.0627x over previous
import jax
from jax.experimental import pallas as pl


_SEQ_BLOCK = 1024


def _add_block(tensor_ref, pos_ref, out_ref):
    i = pl.program_id(0)
    out_ref[...] = tensor_ref[...] + pos_ref[pl.ds(i * _SEQ_BLOCK, _SEQ_BLOCK), :]


def kernel(tensor_in, pos_table):
    batch, seq, dim = tensor_in.shape
    grid = (seq // _SEQ_BLOCK, batch)
    return pl.pallas_call(
        _add_block,
        grid=grid,
        in_specs=[
            pl.BlockSpec((1, _SEQ_BLOCK, dim), lambda i, j: (j, i, 0)),
            pl.BlockSpec((seq, dim), lambda i, j: (0, 0)),
        ],
        out_specs=pl.BlockSpec((1, _SEQ_BLOCK, dim), lambda i, j: (j, i, 0)),
        out_shape=jax.ShapeDtypeStruct(tensor_in.shape, tensor_in.dtype),
    )(tensor_in, pos_table)
